# Initial kernel scaffold; baseline (speedup 1.0000x reference)
import jax
import jax.numpy as jnp
from jax.experimental import pallas as pl


def kernel(x, w1, b1, w2):
    raise NotImplementedError("write your pallas kernel here")


# speedup vs baseline: 4.7307x; 4.7307x over previous
import jax
import jax.numpy as jnp
from jax.experimental import pallas as pl
from jax.experimental.pallas import tpu as pltpu

C = 8
D = H = W = 128
DT = 4
NB = D // DT


def _shift_x(a, dx):
    if dx == 0:
        return a
    z = jnp.zeros(a.shape[:-1] + (1,), a.dtype)
    if dx > 0:
        return jnp.concatenate([a[..., 1:], z], axis=-1)
    return jnp.concatenate([z, a[..., :-1]], axis=-1)


def _shift_y(a, dy):
    if dy == 0:
        return a
    z = jnp.zeros(a.shape[:-2] + (1, a.shape[-1]), a.dtype)
    if dy > 0:
        return jnp.concatenate([a[..., 1:, :], z], axis=-2)
    return jnp.concatenate([z, a[..., :-1, :]], axis=-2)


def _slab_slice(prev_ref, cur_ref, next_ref, s):
    i = pl.program_id(0)
    n = pl.num_programs(0)
    if s == 0:
        v = prev_ref[:, DT - 1]
        v = v * jnp.where(i > 0, 1.0, 0.0).astype(v.dtype)
    elif s <= DT:
        v = cur_ref[:, s - 1]
    else:
        v = next_ref[:, 0]
        v = v * jnp.where(i < n - 1, 1.0, 0.0).astype(v.dtype)
    return v


def _build_shift_ring(g_ref, slot, xs):
    y3 = {dy: _shift_y(xs, dy) for dy in (-1, 0, 1)}
    k = 0
    for dy in (-1, 0, 1):
        for dx in (-1, 0, 1):
            g_ref[slot, pl.ds(k * C, C)] = _shift_x(y3[dy], dx)
            k += 1


def _dot72(wmat, g_ref, slot):
    return jax.lax.dot_general(
        wmat, g_ref[slot], (((1,), (0,)), ((), ())),
        preferred_element_type=jnp.float32)


def _conv1_kernel(xp_ref, xc_ref, xn_ref, w1g_ref, b1_ref, h_ref, g_ref):
    for s in range(DT + 2):
        xs = _slab_slice(xp_ref, xc_ref, xn_ref, s)
        _build_shift_ring(g_ref, s % 3, xs)
        if s >= 2:
            zo = s - 2
            acc = None
            for dz in range(3):
                d = _dot72(w1g_ref[dz], g_ref, (zo + dz) % 3)
                acc = d if acc is None else acc + d
            for co in range(C):
                h_ref[co, zo] = jnp.maximum(acc[co] + b1_ref[0, co], 0.0)


def _conv2_kernel(hp_ref, hc_ref, hn_ref, w2g_ref, w_ref, g_ref):
    for s in range(DT + 2):
        hs = _slab_slice(hp_ref, hc_ref, hn_ref, s)
        _build_shift_ring(g_ref, s % 3, hs)
        if s >= 2:
            zo = s - 2
            acc = None
            for dz in range(3):
                d = _dot72(w2g_ref[dz], g_ref, (zo + dz) % 3)
                acc = d if acc is None else acc + d
            n = jnp.sum(jnp.abs(acc), axis=0)
            r = 1.0 / jnp.maximum(n, 1e-12)
            w_ref[:, zo] = acc * r[None]


def _adapt_kernel(ip_ref, ic_ref, in_ref, w_ref, o_ref, g_ref):
    for s in range(DT + 2):
        vs = _slab_slice(ip_ref, ic_ref, in_ref, s)
        _build_shift_ring(g_ref, s % 3, vs)
        if s >= 2:
            zo = s - 2
            for co in range(C):
                acc = None
                for dz in range(3):
                    slot = (zo + dz) % 3
                    for kk in range(9):
                        tap = dz * 9 + kk
                        t = g_ref[slot, kk * C + co] * w_ref[tap, zo]
                        acc = t if acc is None else acc + t
                o_ref[co, zo] = acc


def _zspec(nch):
    return pl.BlockSpec((nch, DT, H, W), lambda i: (0, i, 0, 0))


def _halo_specs(nch):
    return [
        pl.BlockSpec((nch, DT, H, W), lambda i: (0, jnp.maximum(i - 1, 0), 0, 0)),
        pl.BlockSpec((nch, DT, H, W), lambda i: (0, i, 0, 0)),
        pl.BlockSpec((nch, DT, H, W),
                     lambda i: (0, jnp.minimum(i + 1, NB - 1), 0, 0)),
    ]


def _params(vmem_mb=52):
    return pltpu.CompilerParams(
        dimension_semantics=("parallel",),
        vmem_limit_bytes=vmem_mb * 1024 * 1024,
    )


_RING = pltpu.VMEM((3, 9 * C, H, W), jnp.float32)


def kernel(x, w1, b1, w2):
    xs = x[0]

    w1g = jnp.transpose(w1, (2, 0, 3, 4, 1)).reshape(3, C, 9 * C)
    w2g = jnp.transpose(w2, (2, 0, 3, 4, 1)).reshape(3, 27, 9 * C)
    b1s = b1.reshape(1, C)

    h = pl.pallas_call(
        _conv1_kernel,
        grid=(NB,),
        in_specs=_halo_specs(C) + [
            pl.BlockSpec(memory_space=pltpu.VMEM),
            pl.BlockSpec(memory_space=pltpu.SMEM),
        ],
        out_specs=_zspec(C),
        out_shape=jax.ShapeDtypeStruct((C, D, H, W), jnp.float32),
        scratch_shapes=[_RING],
        compiler_params=_params(),
    )(xs, xs, xs, w1g, b1s)

    wv = pl.pallas_call(
        _conv2_kernel,
        grid=(NB,),
        in_specs=_halo_specs(C) + [pl.BlockSpec(memory_space=pltpu.VMEM)],
        out_specs=_zspec(27),
        out_shape=jax.ShapeDtypeStruct((27, D, H, W), jnp.float32),
        scratch_shapes=[_RING],
        compiler_params=_params(),
    )(h, h, h, w2g)

    out = xs
    for _ in range(3):
        out = pl.pallas_call(
            _adapt_kernel,
            grid=(NB,),
            in_specs=_halo_specs(C) + [_zspec(27)],
            out_specs=_zspec(C),
            out_shape=jax.ShapeDtypeStruct((C, D, H, W), jnp.float32),
            scratch_shapes=[_RING],
            compiler_params=_params(),
        )(out, out, out, wv)

    return out[None]
